# Initial kernel scaffold; baseline (speedup 1.0000x reference)
import jax
import jax.numpy as jnp
from jax.experimental import pallas as pl


def kernel(z):
    raise NotImplementedError("write your pallas kernel here")


# speedup vs baseline: 95.2045x; 95.2045x over previous
import functools

import jax
import jax.numpy as jnp
import numpy as np
from jax import lax
from jax.experimental import pallas as pl
from jax.experimental.pallas import tpu as pltpu
from jax.experimental.pallas import tpu_sc as plsc

_D = 64
_NL = 8
_CHUNK = 128
_STEP = np.float32(2.0 / 6.0)


def _make_rfsq(rows):
    info = plsc.get_sparse_core_info()
    nc, ns, lanes = info.num_cores, info.num_subcores, info.num_lanes
    nw = nc * ns
    rows_per_w = rows // nw
    nchunks = rows_per_w // _CHUNK
    groups = _CHUNK // lanes
    mesh = plsc.VectorSubcoreMesh(core_axis_name="c", subcore_axis_name="s")

    @functools.partial(
        pl.kernel,
        mesh=mesh,
        compiler_params=pltpu.CompilerParams(needs_layout_passes=False),
        out_type=[
            jax.ShapeDtypeStruct((rows * _D,), jnp.float32),
            jax.ShapeDtypeStruct((rows * _D * _NL,), jnp.int32),
        ],
        scratch_types=[
            pltpu.VMEM((_CHUNK * _D,), jnp.float32),
            pltpu.VMEM((_D, _CHUNK), jnp.float32),
            pltpu.VMEM((_CHUNK * _D,), jnp.float32),
            pltpu.VMEM((_CHUNK * _D * _NL,), jnp.int32),
        ],
    )
    def rfsq(z_hbm, qsum_hbm, codes_hbm, zbuf, rT, rrow, cbuf):
        wid = lax.axis_index("s") * nc + lax.axis_index("c")
        row0_w = wid * rows_per_w
        iota = lax.iota(jnp.int32, lanes)

        half = np.float32(0.5)
        three_half = np.float32(1.5)
        magic = np.int32(0x5F3759DF)

        def chunk_body(ci, carry):
            row0 = row0_w + ci * _CHUNK
            pltpu.sync_copy(z_hbm.at[pl.ds(row0 * _D, _CHUNK * _D)], zbuf)

            def group_body(g, gcarry):
                rloc = g * lanes + iota
                rv64 = rloc * _D
                rv512 = rloc * (_D * _NL)

                for j in range(_D):
                    v = plsc.load_gather(zbuf, [rv64 + j])
                    rT[j, pl.ds(g * lanes, lanes)] = v

                def do_layer(l, is_last):
                    s0 = jnp.zeros((lanes,), jnp.float32)
                    s1 = jnp.zeros((lanes,), jnp.float32)
                    q0 = jnp.zeros((lanes,), jnp.float32)
                    q1 = jnp.zeros((lanes,), jnp.float32)
                    for j in range(0, _D, 2):
                        va = rT[j, pl.ds(g * lanes, lanes)]
                        vb = rT[j + 1, pl.ds(g * lanes, lanes)]
                        s0 = s0 + va
                        s1 = s1 + vb
                        q0 = q0 + va * va
                        q1 = q1 + vb * vb
                    mean = (s0 + s1) * np.float32(1.0 / _D)
                    msq = (q0 + q1) * np.float32(1.0 / _D)
                    var = (msq - mean * mean) * np.float32(_D / (_D - 1.0))
                    var = jnp.maximum(var, np.float32(1e-30))
                    bits = lax.bitcast_convert_type(var, jnp.int32)
                    bits = magic - (bits >> 1)
                    y = lax.bitcast_convert_type(bits, jnp.float32)
                    xh = var * half
                    y = y * (three_half - xh * y * y)
                    y = y * (three_half - xh * y * y)
                    y = y * (three_half - xh * y * y)
                    std = var * y + np.float32(1e-5)
                    inv3 = np.float32(3.0) / std
                    c2 = np.float32(3.5) - mean * inv3
                    u = _STEP * std
                    vshift = mean - std

                    for j in range(_D):
                        r = rT[j, pl.ds(g * lanes, lanes)]
                        p = r * inv3 + c2
                        ii = jnp.clip(p.astype(jnp.int32), 0, 6)
                        plsc.store_scatter(cbuf, [(rv512 + j * _NL) + l], ii)
                        zq = ii.astype(jnp.float32) * u + vshift
                        zq_out = r + (zq - r)
                        rn = r - zq_out
                        rT[j, pl.ds(g * lanes, lanes)] = rn
                        if is_last:
                            plsc.store_scatter(rrow, [rv64 + j], rn)

                def layer_loop(l, lcarry):
                    do_layer(l, False)
                    return lcarry

                lax.fori_loop(0, _NL - 1, layer_loop, 0)
                do_layer(_NL - 1, True)
                return gcarry

            lax.fori_loop(0, groups, group_body, 0)

            def qvec_body(i, rcarry):
                a = zbuf[pl.ds(i * lanes, lanes)]
                b = rrow[pl.ds(i * lanes, lanes)]
                zbuf[pl.ds(i * lanes, lanes)] = a - b
                return rcarry

            lax.fori_loop(0, _CHUNK * _D // lanes, qvec_body, 0)
            pltpu.sync_copy(zbuf, qsum_hbm.at[pl.ds(row0 * _D, _CHUNK * _D)])
            pltpu.sync_copy(
                cbuf, codes_hbm.at[pl.ds(row0 * _D * _NL, _CHUNK * _D * _NL)])
            return carry

        lax.fori_loop(0, nchunks, chunk_body, 0)

    return rfsq


def kernel(z):
    b, s, d = z.shape
    rows = b * s
    qsum, codes = _make_rfsq(rows)(z.reshape(rows * d))
    return qsum.reshape(b, s, d), codes.reshape(b, s, d, _NL)


# speedup vs baseline: 95.8334x; 1.0066x over previous
import functools

import jax
import jax.numpy as jnp
import numpy as np
from jax import lax
from jax.experimental import pallas as pl
from jax.experimental.pallas import tpu as pltpu
from jax.experimental.pallas import tpu_sc as plsc

_D = 64
_NL = 8
_CHUNK = 128
_STEP = np.float32(2.0 / 6.0)


def _make_rfsq(rows):
    info = plsc.get_sparse_core_info()
    nc, ns, lanes = info.num_cores, info.num_subcores, info.num_lanes
    nw = nc * ns
    rows_per_w = rows // nw
    nchunks = rows_per_w // _CHUNK
    groups = _CHUNK // lanes
    mesh = plsc.VectorSubcoreMesh(core_axis_name="c", subcore_axis_name="s")

    @functools.partial(
        pl.kernel,
        mesh=mesh,
        compiler_params=pltpu.CompilerParams(needs_layout_passes=False),
        out_type=[
            jax.ShapeDtypeStruct((rows * _D,), jnp.float32),
            jax.ShapeDtypeStruct((rows * _D * _NL,), jnp.int32),
        ],
        scratch_types=[
            pltpu.VMEM((_CHUNK * _D,), jnp.float32),
            pltpu.VMEM((_D, lanes), jnp.float32),
            pltpu.VMEM((_CHUNK * _D,), jnp.float32),
            pltpu.VMEM((_CHUNK * _D * _NL,), jnp.int32),
        ],
    )
    def rfsq(z_hbm, qsum_hbm, codes_hbm, zbuf, rt, qbuf, cbuf):
        wid = lax.axis_index("s") * nc + lax.axis_index("c")
        row0_w = wid * rows_per_w
        iota = lax.iota(jnp.int32, lanes)

        half = np.float32(0.5)
        three_half = np.float32(1.5)
        magic = np.int32(0x5F3759DF)
        zerov = jnp.zeros((lanes,), jnp.float32)

        def chunk_body(ci, carry):
            row0 = row0_w + ci * _CHUNK
            pltpu.sync_copy(z_hbm.at[pl.ds(row0 * _D, _CHUNK * _D)], zbuf)

            def group_body(g, gcarry):
                rloc = g * lanes + iota
                rv64 = rloc * _D
                rv512 = rloc * (_D * _NL)

                s0, s1, q0, q1 = zerov, zerov, zerov, zerov
                for j in range(_D):
                    v = plsc.load_gather(zbuf, [rv64 + j])
                    rt[j] = v
                    if j % 2 == 0:
                        s0 = s0 + v
                        q0 = q0 + v * v
                    else:
                        s1 = s1 + v
                        q1 = q1 + v * v

                def do_layer(l, stats, is_last):
                    s0, s1, q0, q1 = stats
                    mean = (s0 + s1) * np.float32(1.0 / _D)
                    msq = (q0 + q1) * np.float32(1.0 / _D)
                    var = (msq - mean * mean) * np.float32(_D / (_D - 1.0))
                    var = jnp.maximum(var, np.float32(1e-30))
                    bits = lax.bitcast_convert_type(var, jnp.int32)
                    bits = magic - (bits >> 1)
                    y = lax.bitcast_convert_type(bits, jnp.float32)
                    xh = var * half
                    y = y * (three_half - xh * y * y)
                    y = y * (three_half - xh * y * y)
                    y = y * (three_half - xh * y * y)
                    std = var * y + np.float32(1e-5)
                    inv3 = np.float32(3.0) / std
                    c2 = np.float32(3.5) - mean * inv3
                    u = _STEP * std
                    vshift = mean - std
                    rvl = rv512 + l

                    ns0, ns1, nq0, nq1 = zerov, zerov, zerov, zerov
                    for j in range(_D):
                        r = rt[j]
                        p = r * inv3 + c2
                        ii = jnp.clip(p.astype(jnp.int32), 0, 6)
                        plsc.store_scatter(cbuf, [rvl + j * _NL], ii)
                        zq = ii.astype(jnp.float32) * u + vshift
                        zq_out = r + (zq - r)
                        rn = r - zq_out
                        if is_last:
                            zv = plsc.load_gather(zbuf, [rv64 + j])
                            plsc.store_scatter(qbuf, [rv64 + j], zv - rn)
                        else:
                            rt[j] = rn
                            if j % 2 == 0:
                                ns0 = ns0 + rn
                                nq0 = nq0 + rn * rn
                            else:
                                ns1 = ns1 + rn
                                nq1 = nq1 + rn * rn
                    return ns0, ns1, nq0, nq1

                def layer_loop(l, stats):
                    return do_layer(l, stats, False)

                stats = lax.fori_loop(0, _NL - 1, layer_loop, (s0, s1, q0, q1))
                do_layer(_NL - 1, stats, True)
                return gcarry

            lax.fori_loop(0, groups, group_body, 0)

            pltpu.sync_copy(qbuf, qsum_hbm.at[pl.ds(row0 * _D, _CHUNK * _D)])
            pltpu.sync_copy(
                cbuf, codes_hbm.at[pl.ds(row0 * _D * _NL, _CHUNK * _D * _NL)])
            return carry

        lax.fori_loop(0, nchunks, chunk_body, 0)

    return rfsq


def kernel(z):
    b, s, d = z.shape
    rows = b * s
    qsum, codes = _make_rfsq(rows)(z.reshape(rows * d))
    return qsum.reshape(b, s, d), codes.reshape(b, s, d, _NL)
